# baseline (speedup 1.0000x reference)
import functools
import jax
import jax.numpy as jnp
from jax import lax
from jax.experimental import pallas as pl
from jax.experimental.pallas import tpu as pltpu
from jax.experimental.pallas import tpu_sc as plsc

NC, NS, L = 2, 16, 16
NW = NC * NS
R = 8
UNROLL = 4


def kernel(x, pos_table):
    B, S, D = x.shape
    RD = R * D
    s_per_w = S // NW
    n_chunk = s_per_w // R
    xf = x.reshape(B, S * D)
    pf = pos_table.reshape(S * D)
    mesh = plsc.VectorSubcoreMesh(core_axis_name="c", subcore_axis_name="s")

    @functools.partial(
        pl.kernel,
        out_type=jax.ShapeDtypeStruct((B, S * D), jnp.float32),
        mesh=mesh,
        scratch_types=[
            pltpu.VMEM((3, B, RD), jnp.float32),
            pltpu.VMEM((2, RD), jnp.float32),
            pltpu.SemaphoreType.DMA,
            pltpu.SemaphoreType.DMA,
            pltpu.SemaphoreType.DMA,
        ],
    )
    def k(x_hbm, pos_hbm, out_hbm, xb, posb, sem_in, sem_pos, sem_out):
        wid = lax.axis_index("s") * NC + lax.axis_index("c")
        e0 = wid * s_per_w * D

        def start_in(ci):
            base = e0 + ci * RD
            bi = ci % 3
            hs = [
                pltpu.async_copy(
                    x_hbm.at[b, pl.ds(base, RD)], xb.at[bi, b], sem_in)
                for b in range(B)
            ]
            hp = pltpu.async_copy(
                pos_hbm.at[pl.ds(base, RD)], posb.at[ci % 2], sem_pos)
            return hs, hp

        def compute(ci):
            bi = ci % 3
            pb = ci % 2

            def body(i, _):
                for u in range(UNROLL):
                    sl = pl.ds((i * UNROLL + u) * L, L)
                    p = posb[pb, sl]
                    for b in range(B):
                        xb[bi, b, sl] = xb[bi, b, sl] + p
                return ()

            lax.fori_loop(0, RD // L // UNROLL, body, ())

        def start_out(ci):
            base = e0 + ci * RD
            bi = ci % 3
            return [
                pltpu.async_copy(
                    xb.at[bi, b], out_hbm.at[b, pl.ds(base, RD)], sem_out)
                for b in range(B)
            ]

        in_h = [None] * n_chunk
        pos_h = [None] * n_chunk
        out_h = [None] * n_chunk
        in_h[0], pos_h[0] = start_in(0)
        for ci in range(n_chunk):
            if ci >= 2:
                for h in out_h[ci - 2]:
                    h.wait()
            if ci + 1 < n_chunk:
                in_h[ci + 1], pos_h[ci + 1] = start_in(ci + 1)
            for h in in_h[ci]:
                h.wait()
            pos_h[ci].wait()
            compute(ci)
            out_h[ci] = start_out(ci)
        for ci in range(max(n_chunk - 2, 0), n_chunk):
            for h in out_h[ci]:
                h.wait()

    return k(xf, pf).reshape(B, S, D)


# speedup vs baseline: 1.4762x; 1.4762x over previous
import functools
import jax
import jax.numpy as jnp
from jax import lax
from jax.experimental import pallas as pl
from jax.experimental.pallas import tpu as pltpu
from jax.experimental.pallas import tpu_sc as plsc

NC, NS, L = 2, 16, 16
NW = NC * NS
R = 8
UNROLL = 16


def kernel(x, pos_table):
    B, S, D = x.shape
    RD = R * D
    s_per_w = S // NW
    n_chunk = s_per_w // R
    xf = x.reshape(B, S * D)
    pf = pos_table.reshape(S * D)
    mesh = plsc.VectorSubcoreMesh(core_axis_name="c", subcore_axis_name="s")

    @functools.partial(
        pl.kernel,
        out_type=jax.ShapeDtypeStruct((B, S * D), jnp.float32),
        mesh=mesh,
        scratch_types=[
            pltpu.VMEM((3, B, RD), jnp.float32),
            pltpu.VMEM((2, RD), jnp.float32),
            pltpu.SemaphoreType.DMA,
            pltpu.SemaphoreType.DMA,
            pltpu.SemaphoreType.DMA,
        ],
    )
    def k(x_hbm, pos_hbm, out_hbm, xb, posb, sem_in, sem_pos, sem_out):
        wid = lax.axis_index("s") * NC + lax.axis_index("c")
        e0 = wid * s_per_w * D

        def start_in(ci):
            base = e0 + ci * RD
            bi = ci % 3
            hs = [
                pltpu.async_copy(
                    x_hbm.at[b, pl.ds(base, RD)], xb.at[bi, b], sem_in)
                for b in range(B)
            ]
            hp = pltpu.async_copy(
                pos_hbm.at[pl.ds(base, RD)], posb.at[ci % 2], sem_pos)
            return hs, hp

        def compute(ci):
            bi = ci % 3
            pb = ci % 2

            def body(i, _):
                base = i * UNROLL * L
                ps = [posb[pb, pl.ds(base + u * L, L)] for u in range(UNROLL)]
                for b in range(B):
                    for u in range(UNROLL):
                        sl = pl.ds(base + u * L, L)
                        xb[bi, b, sl] = xb[bi, b, sl] + ps[u]
                return ()

            lax.fori_loop(0, RD // L // UNROLL, body, ())

        def start_out(ci):
            base = e0 + ci * RD
            bi = ci % 3
            return [
                pltpu.async_copy(
                    xb.at[bi, b], out_hbm.at[b, pl.ds(base, RD)], sem_out)
                for b in range(B)
            ]

        in_h = [None] * n_chunk
        pos_h = [None] * n_chunk
        out_h = [None] * n_chunk
        in_h[0], pos_h[0] = start_in(0)
        for ci in range(n_chunk):
            if ci >= 2:
                for h in out_h[ci - 2]:
                    h.wait()
            if ci + 1 < n_chunk:
                in_h[ci + 1], pos_h[ci + 1] = start_in(ci + 1)
            for h in in_h[ci]:
                h.wait()
            pos_h[ci].wait()
            compute(ci)
            out_h[ci] = start_out(ci)
        for ci in range(max(n_chunk - 2, 0), n_chunk):
            for h in out_h[ci]:
                h.wait()

    return k(xf, pf).reshape(B, S, D)


# speedup vs baseline: 6.2591x; 4.2400x over previous
import jax
import jax.numpy as jnp
from jax.experimental import pallas as pl
from jax.experimental.pallas import tpu as pltpu

SEQ_BLOCK = 512


def _add_kernel(x_ref, pos_ref, o_ref):
    o_ref[...] = x_ref[...] + pos_ref[...]


def kernel(x, pos_table):
    B, S, D = x.shape
    num_s = S // SEQ_BLOCK
    return pl.pallas_call(
        _add_kernel,
        grid=(num_s,),
        in_specs=[
            pl.BlockSpec((B, SEQ_BLOCK, D), lambda s: (0, s, 0)),
            pl.BlockSpec((SEQ_BLOCK, D), lambda s: (s, 0)),
        ],
        out_specs=pl.BlockSpec((B, SEQ_BLOCK, D), lambda s: (0, s, 0)),
        out_shape=jax.ShapeDtypeStruct((B, S, D), x.dtype),
    )(x, pos_table)
